# baseline (speedup 1.0000x reference)
import jax
import jax.numpy as jnp
from jax import lax
from jax.experimental import pallas as pl
from jax.experimental.pallas import tpu as pltpu
from jax.experimental.pallas import tpu_sc as plsc

F32 = jnp.float32
I32 = jnp.int32

_NC = 2
_NS = 16
_L = 128
_G = 8
_SPLIT = (48, 112)


def _sc_aggregate(x, src2d, dst2d, attr2d, n_nodes, nch0, nch1):
    d = x.shape[1]
    acc_rows_per_tile = (((n_nodes + 1 + _NS - 1) // _NS) + 7) // 8 * 8
    n_acc = acc_rows_per_tile * _NS
    rows_out = n_nodes // _NS // 8 * 8
    rows_tail = n_nodes - _NS * rows_out

    mesh = plsc.VectorSubcoreMesh(core_axis_name="c", subcore_axis_name="s")

    def body(x_hbm, src_hbm, dst_hbm, attr_hbm, out_hbm,
             src_a, src_b, dst_a, dst_b, attr_a, attr_b, rows_a, rows_b,
             acc, sem_g, sem_s, sem_ia, sem_ib):
        c = lax.axis_index("c")
        s = lax.axis_index("s")
        rows_bufs = (rows_a, rows_b)
        src_bufs = (src_a, src_b)
        dst_bufs = (dst_a, dst_b)
        attr_bufs = (attr_a, attr_b)
        sem_i = (sem_ia, sem_ib)

        nch_c = jnp.where(c == 0, nch0, nch1)
        base = jnp.where(c == 0, s * nch0, _NS * nch0 + s * nch1)
        npairs = jnp.where(c == 0, nch0 // (2 * _G), nch1 // (2 * _G))

        zero16 = jnp.zeros((16,), F32)

        def zrow(r, carry):
            for j in range(d // 16):
                rows_a[r, pl.ds(16 * j, 16)] = zero16
            return carry

        lax.fori_loop(0, _L, zrow, 0)

        nfull = acc_rows_per_tile // _L
        def zcopy(i, carry):
            pltpu.sync_copy(
                rows_a, acc.at[pl.ds(s * acc_rows_per_tile + i * _L, _L)])
            return carry

        lax.fori_loop(0, nfull, zcopy, 0)
        ztail = acc_rows_per_tile - nfull * _L
        if ztail:
            pltpu.sync_copy(
                rows_a.at[pl.ds(0, ztail)],
                acc.at[pl.ds(s * acc_rows_per_tile + nfull * _L, ztail)])
        plsc.subcore_barrier()

        def scale(buf, attr_v, b):
            def grp16(k, icarry):
                att16 = attr_v[pl.ds(b * _L + k * 16, 16)]
                for el in range(16):
                    a = jnp.full((16,), att16[el], F32)
                    row = k * 16 + el
                    for j in range(d // 16):
                        buf[row, pl.ds(16 * j, 16)] = (
                            buf[row, pl.ds(16 * j, 16)] * a)
                return icarry

            lax.fori_loop(0, _L // 16, grp16, 0)

        def idx_copies(par, g):
            goff = base + g * _G
            return (
                pltpu.make_async_copy(src_hbm.at[pl.ds(goff, _G)],
                                      src_bufs[par], sem_i[par]),
                pltpu.make_async_copy(dst_hbm.at[pl.ds(goff, _G)],
                                      dst_bufs[par], sem_i[par]),
                pltpu.make_async_copy(
                    attr_hbm.at[pl.ds(goff * _L, _G * _L)],
                    attr_bufs[par].at[pl.ds(0, _G * _L)], sem_i[par]),
            )

        def stage_idx(par, g):
            for cp in idx_copies(par, g):
                cp.start()

        def wait_idx(par, g):
            for cp in idx_copies(par, g):
                cp.wait()

        def group(g, par):
            src_v, dst_v = src_bufs[par], dst_bufs[par]
            wait_idx(par, g)
            gathers = [None] * _G
            scatters = [None] * _G
            gathers[0] = pltpu.async_copy(
                x_hbm.at[src_v.at[0]], rows_bufs[0], sem_g)
            for b in range(_G):
                cur = rows_bufs[b % 2]
                gathers[b].wait()
                if b + 1 < _G:
                    if b >= 1:
                        scatters[b - 1].wait()
                    gathers[b + 1] = pltpu.async_copy(
                        x_hbm.at[src_v.at[b + 1]], rows_bufs[(b + 1) % 2],
                        sem_g)
                scale(cur, attr_bufs[par], b)
                scatters[b] = pltpu.async_copy(
                    cur, acc.at[dst_v.at[b]], sem_s, add=True)
            scatters[_G - 2].wait()
            scatters[_G - 1].wait()
            @pl.when(g + 2 < 2 * npairs)
            def _():
                stage_idx(par, g + 2)

        stage_idx(0, 0)
        stage_idx(1, 1)

        def group_pair(gp, carry):
            group(2 * gp, 0)
            group(2 * gp + 1, 1)
            return carry

        lax.fori_loop(0, npairs, group_pair, 0)
        plsc.subcore_barrier()

        pltpu.sync_copy(acc.at[pl.ds(s * rows_out, rows_out)],
                        out_hbm.at[c, pl.ds(s * rows_out, rows_out)])
        if rows_tail:
            @pl.when(s == _NS - 1)
            def _tail():
                pltpu.sync_copy(
                    acc.at[pl.ds(_NS * rows_out, rows_tail)],
                    out_hbm.at[c, pl.ds(_NS * rows_out, rows_tail)])

    return pl.kernel(
        body,
        out_type=jax.ShapeDtypeStruct((_NC, n_nodes, d), F32),
        mesh=mesh,
        scratch_types=[
            pltpu.VMEM((_G, _L), I32),
            pltpu.VMEM((_G, _L), I32),
            pltpu.VMEM((_G, _L), I32),
            pltpu.VMEM((_G, _L), I32),
            pltpu.VMEM((_G * _L + 16,), F32),
            pltpu.VMEM((_G * _L + 16,), F32),
            pltpu.VMEM((_L, d), F32),
            pltpu.VMEM((_L, d), F32),
            pltpu.VMEM_SHARED((n_acc, d), F32),
            pltpu.SemaphoreType.DMA,
            pltpu.SemaphoreType.DMA,
            pltpu.SemaphoreType.DMA,
            pltpu.SemaphoreType.DMA,
        ],
    )(x, src2d, dst2d, attr2d)


def _combine_matmul(partials, w, bias2d):
    n, d = partials.shape[1], partials.shape[2]
    d_out = w.shape[1]
    blk = 1000
    grid = n // blk

    def body(p_ref, w_ref, b_ref, o_ref):
        a = p_ref[0] + p_ref[1]
        o_ref[...] = (jnp.dot(a, w_ref[...], preferred_element_type=F32)
                      + b_ref[...])

    return pl.pallas_call(
        body,
        grid=(grid,),
        in_specs=[
            pl.BlockSpec((2, blk, d), lambda i: (0, i, 0)),
            pl.BlockSpec((d, d_out), lambda i: (0, 0)),
            pl.BlockSpec((1, d_out), lambda i: (0, 0)),
        ],
        out_specs=pl.BlockSpec((blk, d_out), lambda i: (i, 0)),
        out_shape=jax.ShapeDtypeStruct((n, d_out), F32),
    )(partials, w, bias2d)


def kernel(x, edge_indices, edge_attr, kernel, bias):
    n, _ = x.shape
    e = edge_attr.shape[0]
    dst = edge_indices[0].astype(I32)
    src = edge_indices[1].astype(I32)
    attr = edge_attr.astype(F32)

    gran = 2 * _G
    total = _NS * (_SPLIT[0] + _SPLIT[1])
    need = (e + _L - 1) // _L
    assert need <= total
    nch0, nch1 = _SPLIT
    e_pad = total * _L
    pad = e_pad - e
    if pad:
        src = jnp.concatenate([src, jnp.zeros((pad,), I32)])
        dst = jnp.concatenate([dst, jnp.full((pad,), n, I32)])
        attr = jnp.concatenate([attr, jnp.zeros((pad,), F32)])

    partials = _sc_aggregate(x, src.reshape(-1, _L), dst.reshape(-1, _L),
                             attr, n, nch0, nch1)
    return _combine_matmul(partials, kernel, bias.reshape(1, -1))


# speedup vs baseline: 1.1303x; 1.1303x over previous
import jax
import jax.numpy as jnp
from jax import lax
from jax.experimental import pallas as pl
from jax.experimental.pallas import tpu as pltpu
from jax.experimental.pallas import tpu_sc as plsc

F32 = jnp.float32
I32 = jnp.int32

_NC = 2
_NS = 16
_L = 128
_G = 8
_SPLIT = (112, 48)


def _sc_aggregate(x, src2d, dst2d, attr2d, n_nodes, nch0, nch1):
    d = x.shape[1]
    acc_rows_per_tile = (((n_nodes + 1 + _NS - 1) // _NS) + 7) // 8 * 8
    n_acc = acc_rows_per_tile * _NS
    rows_out = n_nodes // _NS // 8 * 8
    rows_tail = n_nodes - _NS * rows_out

    mesh = plsc.VectorSubcoreMesh(core_axis_name="c", subcore_axis_name="s")

    def body(x_hbm, src_hbm, dst_hbm, attr_hbm, out_hbm,
             src_a, src_b, dst_a, dst_b, attr_a, attr_b, rows_a, rows_b,
             acc, sem_g, sem_s, sem_ia, sem_ib):
        c = lax.axis_index("c")
        s = lax.axis_index("s")
        rows_bufs = (rows_a, rows_b)
        src_bufs = (src_a, src_b)
        dst_bufs = (dst_a, dst_b)
        attr_bufs = (attr_a, attr_b)
        sem_i = (sem_ia, sem_ib)

        nch_c = jnp.where(c == 0, nch0, nch1)
        base = jnp.where(c == 0, s * nch0, _NS * nch0 + s * nch1)
        npairs = jnp.where(c == 0, nch0 // (2 * _G), nch1 // (2 * _G))

        zero16 = jnp.zeros((16,), F32)

        def zrow(r, carry):
            for j in range(d // 16):
                rows_a[r, pl.ds(16 * j, 16)] = zero16
            return carry

        lax.fori_loop(0, _L, zrow, 0)

        nfull = acc_rows_per_tile // _L
        def zcopy(i, carry):
            pltpu.sync_copy(
                rows_a, acc.at[pl.ds(s * acc_rows_per_tile + i * _L, _L)])
            return carry

        lax.fori_loop(0, nfull, zcopy, 0)
        ztail = acc_rows_per_tile - nfull * _L
        if ztail:
            pltpu.sync_copy(
                rows_a.at[pl.ds(0, ztail)],
                acc.at[pl.ds(s * acc_rows_per_tile + nfull * _L, ztail)])
        plsc.subcore_barrier()

        def scale(buf, attr_v, b):
            def grp16(k, icarry):
                att16 = attr_v[pl.ds(b * _L + k * 16, 16)]
                for el in range(16):
                    a = jnp.full((16,), att16[el], F32)
                    row = k * 16 + el
                    for j in range(d // 16):
                        buf[row, pl.ds(16 * j, 16)] = (
                            buf[row, pl.ds(16 * j, 16)] * a)
                return icarry

            lax.fori_loop(0, _L // 16, grp16, 0)

        def idx_copies(par, g):
            goff = base + g * _G
            return (
                pltpu.make_async_copy(src_hbm.at[pl.ds(goff, _G)],
                                      src_bufs[par], sem_i[par]),
                pltpu.make_async_copy(dst_hbm.at[pl.ds(goff, _G)],
                                      dst_bufs[par], sem_i[par]),
                pltpu.make_async_copy(
                    attr_hbm.at[pl.ds(goff * _L, _G * _L)],
                    attr_bufs[par].at[pl.ds(0, _G * _L)], sem_i[par]),
            )

        def stage_idx(par, g):
            for cp in idx_copies(par, g):
                cp.start()

        def wait_idx(par, g):
            for cp in idx_copies(par, g):
                cp.wait()

        def group(g, par):
            src_v, dst_v = src_bufs[par], dst_bufs[par]
            wait_idx(par, g)
            gathers = [None] * _G
            scatters = [None] * _G
            gathers[0] = pltpu.async_copy(
                x_hbm.at[src_v.at[0]], rows_bufs[0], sem_g)
            for b in range(_G):
                cur = rows_bufs[b % 2]
                gathers[b].wait()
                if b + 1 < _G:
                    if b >= 1:
                        scatters[b - 1].wait()
                    gathers[b + 1] = pltpu.async_copy(
                        x_hbm.at[src_v.at[b + 1]], rows_bufs[(b + 1) % 2],
                        sem_g)
                scale(cur, attr_bufs[par], b)
                scatters[b] = pltpu.async_copy(
                    cur, acc.at[dst_v.at[b]], sem_s, add=True)
            scatters[_G - 2].wait()
            scatters[_G - 1].wait()
            @pl.when(g + 2 < 2 * npairs)
            def _():
                stage_idx(par, g + 2)

        stage_idx(0, 0)
        stage_idx(1, 1)

        def group_pair(gp, carry):
            group(2 * gp, 0)
            group(2 * gp + 1, 1)
            return carry

        lax.fori_loop(0, npairs, group_pair, 0)
        plsc.subcore_barrier()

        pltpu.sync_copy(acc.at[pl.ds(s * rows_out, rows_out)],
                        out_hbm.at[c, pl.ds(s * rows_out, rows_out)])
        if rows_tail:
            @pl.when(s == _NS - 1)
            def _tail():
                pltpu.sync_copy(
                    acc.at[pl.ds(_NS * rows_out, rows_tail)],
                    out_hbm.at[c, pl.ds(_NS * rows_out, rows_tail)])

    return pl.kernel(
        body,
        out_type=jax.ShapeDtypeStruct((_NC, n_nodes, d), F32),
        mesh=mesh,
        scratch_types=[
            pltpu.VMEM((_G, _L), I32),
            pltpu.VMEM((_G, _L), I32),
            pltpu.VMEM((_G, _L), I32),
            pltpu.VMEM((_G, _L), I32),
            pltpu.VMEM((_G * _L + 16,), F32),
            pltpu.VMEM((_G * _L + 16,), F32),
            pltpu.VMEM((_L, d), F32),
            pltpu.VMEM((_L, d), F32),
            pltpu.VMEM_SHARED((n_acc, d), F32),
            pltpu.SemaphoreType.DMA,
            pltpu.SemaphoreType.DMA,
            pltpu.SemaphoreType.DMA,
            pltpu.SemaphoreType.DMA,
        ],
    )(x, src2d, dst2d, attr2d)


def _combine_matmul(partials, w, bias2d):
    n, d = partials.shape[1], partials.shape[2]
    d_out = w.shape[1]
    blk = 1000
    grid = n // blk

    def body(p_ref, w_ref, b_ref, o_ref):
        a = p_ref[0] + p_ref[1]
        o_ref[...] = (jnp.dot(a, w_ref[...], preferred_element_type=F32)
                      + b_ref[...])

    return pl.pallas_call(
        body,
        grid=(grid,),
        in_specs=[
            pl.BlockSpec((2, blk, d), lambda i: (0, i, 0)),
            pl.BlockSpec((d, d_out), lambda i: (0, 0)),
            pl.BlockSpec((1, d_out), lambda i: (0, 0)),
        ],
        out_specs=pl.BlockSpec((blk, d_out), lambda i: (i, 0)),
        out_shape=jax.ShapeDtypeStruct((n, d_out), F32),
    )(partials, w, bias2d)


def kernel(x, edge_indices, edge_attr, kernel, bias):
    n, _ = x.shape
    e = edge_attr.shape[0]
    dst = edge_indices[0].astype(I32)
    src = edge_indices[1].astype(I32)
    attr = edge_attr.astype(F32)

    gran = 2 * _G
    total = _NS * (_SPLIT[0] + _SPLIT[1])
    need = (e + _L - 1) // _L
    assert need <= total
    nch0, nch1 = _SPLIT
    e_pad = total * _L
    pad = e_pad - e
    if pad:
        src = jnp.concatenate([src, jnp.zeros((pad,), I32)])
        dst = jnp.concatenate([dst, jnp.full((pad,), n, I32)])
        attr = jnp.concatenate([attr, jnp.zeros((pad,), F32)])

    partials = _sc_aggregate(x, src.reshape(-1, _L), dst.reshape(-1, _L),
                             attr, n, nch0, nch1)
    return _combine_matmul(partials, kernel, bias.reshape(1, -1))
